# baseline (speedup 1.0000x reference)
import functools

import jax
import jax.numpy as jnp
from jax import lax
from jax.experimental import pallas as pl
from jax.experimental.pallas import tpu as pltpu
from jax.experimental.pallas import tpu_sc as plsc

N = 10000
E = 320000
D = 128
NC = 2
NS = 16
NW = NC * NS
EPW = E // NW
K = 128
FULL = EPW // K
TAIL = EPW - FULL * K
RPT = N // NS

BM = 1000
GRID = N // BM

_MESH = plsc.VectorSubcoreMesh(
    core_axis_name="c", subcore_axis_name="s", num_cores=NC, num_subcores=NS
)

_DEG_CHUNK = 632
_DEG_LAST = N - (NS - 1) * _DEG_CHUNK


@functools.partial(
    pl.kernel,
    mesh=_MESH,
    out_type=jax.ShapeDtypeStruct((NC * N,), jnp.float32),
    scratch_types=[
        pltpu.VMEM((K,), jnp.float32),
        pltpu.VMEM((K,), jnp.int32),
        pltpu.VMEM((TAIL,), jnp.int32),
        pltpu.VMEM((_DEG_CHUNK + 8,), jnp.float32),
        pltpu.VMEM_SHARED((N,), jnp.float32),
        pltpu.SemaphoreType.DMA,
    ],
)
def _deg_kernel(dst_hbm, out_hbm, ones_v, dst_v, dst_t, zbuf, acc_sh, sem):
    c = lax.axis_index("c")
    s = lax.axis_index("s")
    wid = c * NS + s
    for j in range(K // 16):
        ones_v[pl.ds(j * 16, 16)] = jnp.ones((16,), jnp.float32)
    for j in range((_DEG_CHUNK + 8) // 16):
        zbuf[pl.ds(j * 16, 16)] = jnp.zeros((16,), jnp.float32)

    @pl.when(s < NS - 1)
    def _():
        o = s * _DEG_CHUNK
        pltpu.sync_copy(zbuf.at[pl.ds(0, _DEG_CHUNK)], acc_sh.at[pl.ds(o, _DEG_CHUNK)])

    @pl.when(s == NS - 1)
    def _():
        o = (NS - 1) * _DEG_CHUNK
        pltpu.sync_copy(zbuf.at[pl.ds(0, _DEG_LAST)], acc_sh.at[pl.ds(o, _DEG_LAST)])

    plsc.subcore_barrier()
    base = pl.multiple_of(wid * EPW, 8)

    @pl.loop(0, FULL)
    def _(g):
        off = pl.multiple_of(base + g * K, 8)
        pltpu.sync_copy(dst_hbm.at[pl.ds(off, K)], dst_v)
        pltpu.sync_copy(ones_v, acc_sh.at[dst_v], add=True)

    offt = pl.multiple_of(base + FULL * K, 8)
    pltpu.sync_copy(dst_hbm.at[pl.ds(offt, TAIL)], dst_t)
    pltpu.sync_copy(ones_v.at[pl.ds(0, TAIL)], acc_sh.at[dst_t], add=True)
    plsc.subcore_barrier()

    cbase = pl.multiple_of(c * N, 8)

    @pl.when(s < NS - 1)
    def _():
        o = s * _DEG_CHUNK
        pltpu.sync_copy(acc_sh.at[pl.ds(o, _DEG_CHUNK)], zbuf.at[pl.ds(0, _DEG_CHUNK)])
        pltpu.sync_copy(
            zbuf.at[pl.ds(0, _DEG_CHUNK)], out_hbm.at[pl.ds(cbase + o, _DEG_CHUNK)]
        )

    @pl.when(s == NS - 1)
    def _():
        o = (NS - 1) * _DEG_CHUNK
        pltpu.sync_copy(acc_sh.at[pl.ds(o, _DEG_LAST)], zbuf.at[pl.ds(0, _DEG_LAST)])
        pltpu.sync_copy(
            zbuf.at[pl.ds(0, _DEG_LAST)], out_hbm.at[pl.ds(cbase + o, _DEG_LAST)]
        )


RANGE = N // NC
NA = 5072
EPT = E // NS
FULL2 = EPT // K
TAIL2 = EPT - FULL2 * K
SB_G = 12
SBE = SB_G * K
NSB = FULL2 // SB_G
NBUF = 2
_RW_CHUNK = 312
_RW_LAST = RANGE - (NS - 1) * _RW_CHUNK


def _remap_chunk(dst_ref, src_off, fdst_ref, j, k, lo):
    t = dst_ref[pl.ds(src_off + j * K + k * 16, 16)] - lo
    oob = plsc.bitcast(t, jnp.uint32) >= jnp.uint32(RANGE)
    trash = RANGE + (t & 63)
    fdst_ref[j, pl.ds(k * 16, 16)] = jnp.where(oob, trash, t)


@functools.partial(
    pl.kernel,
    mesh=_MESH,
    out_type=jax.ShapeDtypeStruct((NC, RANGE, D), jnp.float32),
    scratch_types=[
        [pltpu.VMEM((SBE,), jnp.int32) for _ in range(2)],
        [pltpu.VMEM((SBE,), jnp.int32) for _ in range(2)],
        [pltpu.VMEM((SB_G, K), jnp.int32) for _ in range(1)],
        pltpu.VMEM((NBUF * K, D), jnp.float32),
        pltpu.VMEM((TAIL2,), jnp.int32),
        pltpu.VMEM((TAIL2,), jnp.int32),
        pltpu.VMEM((1, TAIL2), jnp.int32),
        pltpu.VMEM((TAIL2, D), jnp.float32),
        pltpu.VMEM((_RW_LAST, D), jnp.float32),
        pltpu.VMEM_SHARED((NA, D), jnp.float32),
        pltpu.SemaphoreType.DMA,
        pltpu.SemaphoreType.DMA,
        pltpu.SemaphoreType.DMA,
    ],
)
def _agg_kernel(
    h_hbm, src_hbm, dst_hbm, out_hbm,
    srcb, dstb, fdstb, rows, src_t, dst_t, fdst_t, rows_t, bounce, acc_sh,
    gsem, sem, isem,
):
    c = lax.axis_index("c")
    s = lax.axis_index("s")
    lo = c * RANGE

    @pl.when(s < NS - 1)
    def _():
        r0 = s * _RW_CHUNK
        pltpu.sync_copy(h_hbm.at[pl.ds(lo + r0, _RW_CHUNK)], bounce.at[pl.ds(0, _RW_CHUNK)])
        pltpu.sync_copy(bounce.at[pl.ds(0, _RW_CHUNK)], acc_sh.at[pl.ds(r0, _RW_CHUNK)])

    @pl.when(s == NS - 1)
    def _():
        r0 = (NS - 1) * _RW_CHUNK
        pltpu.sync_copy(h_hbm.at[pl.ds(lo + r0, _RW_LAST)], bounce.at[pl.ds(0, _RW_LAST)])
        pltpu.sync_copy(bounce.at[pl.ds(0, _RW_LAST)], acc_sh.at[pl.ds(r0, _RW_LAST)])

    plsc.subcore_barrier()

    base = pl.multiple_of(s * EPT, 8)

    def _sb_issue(set_, sb):
        off = pl.multiple_of(base + sb * SBE, 8)
        pltpu.async_copy(src_hbm.at[pl.ds(off, SBE)], srcb[set_], isem)
        pltpu.async_copy(dst_hbm.at[pl.ds(off, SBE)], dstb[set_], isem)

    def _sb_wait(set_, sb):
        off = pl.multiple_of(base + sb * SBE, 8)
        pltpu.make_async_copy(src_hbm.at[pl.ds(off, SBE)], srcb[set_], isem).wait()
        pltpu.make_async_copy(dst_hbm.at[pl.ds(off, SBE)], dstb[set_], isem).wait()

    def _sb_exec(set_, sb):
        for j in range(SB_G):
            for k in range(K // 16):
                _remap_chunk(dstb[set_], 0, fdstb[0], j, k, lo)

        @pl.loop(0, SB_G)
        def _(j):
            b = (j % NBUF) * K
            pltpu.async_copy(
                h_hbm.at[srcb[set_].at[pl.ds(j * K, K)]], rows.at[pl.ds(b, K)], gsem
            )

            @pl.when(j > 0)
            def _():
                jm = j - 1
                bm = (jm % NBUF) * K
                pltpu.make_async_copy(
                    h_hbm.at[srcb[set_].at[pl.ds(jm * K, K)]],
                    rows.at[pl.ds(bm, K)], gsem,
                ).wait()
                pltpu.sync_copy(rows.at[pl.ds(bm, K)], acc_sh.at[fdstb[0].at[jm]], add=True)

        jl = SB_G - 1
        bl = (jl % NBUF) * K
        pltpu.make_async_copy(
            h_hbm.at[srcb[set_].at[pl.ds(jl * K, K)]], rows.at[pl.ds(bl, K)], gsem
        ).wait()
        pltpu.sync_copy(rows.at[pl.ds(bl, K)], acc_sh.at[fdstb[0].at[jl]], add=True)

    _sb_issue(0, 0)

    @pl.loop(0, (NSB - 1) // 2)
    def _(i):
        sb0 = i * 2
        _sb_wait(0, sb0)
        _sb_issue(1, sb0 + 1)
        _sb_exec(0, sb0)
        _sb_wait(1, sb0 + 1)
        _sb_issue(0, sb0 + 2)
        _sb_exec(1, sb0 + 1)

    _sb_wait(0, NSB - 1)
    _sb_exec(0, NSB - 1)

    offt = pl.multiple_of(base + FULL2 * K, 8)
    pltpu.sync_copy(src_hbm.at[pl.ds(offt, TAIL2)], src_t)
    pltpu.sync_copy(dst_hbm.at[pl.ds(offt, TAIL2)], dst_t)
    for k in range(TAIL2 // 16):
        t = dst_t[pl.ds(k * 16, 16)] - lo
        oob = plsc.bitcast(t, jnp.uint32) >= jnp.uint32(RANGE)
        trash = RANGE + (t & 63)
        fdst_t[0, pl.ds(k * 16, 16)] = jnp.where(oob, trash, t)
    pltpu.async_copy(h_hbm.at[src_t], rows_t, sem).wait()
    pltpu.sync_copy(rows_t, acc_sh.at[fdst_t.at[0]], add=True)

    plsc.subcore_barrier()

    @pl.when(s < NS - 1)
    def _():
        r0 = s * _RW_CHUNK
        pltpu.sync_copy(acc_sh.at[pl.ds(r0, _RW_CHUNK)], bounce.at[pl.ds(0, _RW_CHUNK)])
        pltpu.sync_copy(bounce.at[pl.ds(0, _RW_CHUNK)], out_hbm.at[c, pl.ds(r0, _RW_CHUNK)])

    @pl.when(s == NS - 1)
    def _():
        r0 = (NS - 1) * _RW_CHUNK
        pltpu.sync_copy(acc_sh.at[pl.ds(r0, _RW_LAST)], bounce.at[pl.ds(0, _RW_LAST)])
        pltpu.sync_copy(bounce.at[pl.ds(0, _RW_LAST)], out_hbm.at[c, pl.ds(r0, _RW_LAST)])


def _mm_body(x_ref, w_ref, o_ref):
    o_ref[...] = jnp.dot(
        x_ref[...], w_ref[...], preferred_element_type=jnp.float32
    )


_mm = pl.pallas_call(
    _mm_body,
    grid=(GRID,),
    in_specs=[
        pl.BlockSpec((BM, D), lambda i: (i, 0)),
        pl.BlockSpec((D, D), lambda i: (0, 0)),
    ],
    out_specs=pl.BlockSpec((BM, D), lambda i: (i, 0)),
    out_shape=jax.ShapeDtypeStruct((N, D), jnp.float32),
)


def _scale_body(h_ref, degp_ref, hp_ref, dinv_ref):
    dv = lax.rsqrt(degp_ref[0] + degp_ref[1] + 1.0)
    dinv_ref[...] = dv
    hp_ref[...] = dv * h_ref[...]


_scale = pl.pallas_call(
    _scale_body,
    grid=(GRID,),
    in_specs=[
        pl.BlockSpec((BM, D), lambda i: (i, 0)),
        pl.BlockSpec((2, BM, 1), lambda i: (0, i, 0)),
    ],
    out_specs=[
        pl.BlockSpec((BM, D), lambda i: (i, 0)),
        pl.BlockSpec((BM, 1), lambda i: (i, 0)),
    ],
    out_shape=[
        jax.ShapeDtypeStruct((N, D), jnp.float32),
        jax.ShapeDtypeStruct((N, 1), jnp.float32),
    ],
)


def _mid_body(a_ref, dinv_ref, b_ref, w_ref, o_ref):
    dv = dinv_ref[...]
    z = jnp.maximum(dv * a_ref[...] + b_ref[...], 0.0)
    o_ref[...] = dv * jnp.dot(z, w_ref[...], preferred_element_type=jnp.float32)


_mid = pl.pallas_call(
    _mid_body,
    grid=(GRID,),
    in_specs=[
        pl.BlockSpec((BM, D), lambda i: (i, 0)),
        pl.BlockSpec((BM, 1), lambda i: (i, 0)),
        pl.BlockSpec((1, D), lambda i: (0, 0)),
        pl.BlockSpec((D, D), lambda i: (0, 0)),
    ],
    out_specs=pl.BlockSpec((BM, D), lambda i: (i, 0)),
    out_shape=jax.ShapeDtypeStruct((N, D), jnp.float32),
)


def _final_body(a_ref, dinv_ref, b_ref, o_ref):
    o_ref[...] = dinv_ref[...] * a_ref[...] + b_ref[...]


_final = pl.pallas_call(
    _final_body,
    grid=(GRID,),
    in_specs=[
        pl.BlockSpec((BM, D), lambda i: (i, 0)),
        pl.BlockSpec((BM, 1), lambda i: (i, 0)),
        pl.BlockSpec((1, D), lambda i: (0, 0)),
    ],
    out_specs=pl.BlockSpec((BM, D), lambda i: (i, 0)),
    out_shape=jax.ShapeDtypeStruct((N, D), jnp.float32),
)


def kernel(x, edge_index, W1, b1, W2, b2):
    src = edge_index[0].astype(jnp.int32)
    dst = edge_index[1].astype(jnp.int32)

    degp = _deg_kernel(dst)
    h1 = _mm(x, W1)
    h1p, dinv = _scale(h1, degp.reshape(2, N, 1))
    a1 = _agg_kernel(h1p, src, dst).reshape(N, D)
    h2p = _mid(a1, dinv, b1.reshape(1, D), W2)
    a2 = _agg_kernel(h2p, src, dst).reshape(N, D)
    return _final(a2, dinv, b2.reshape(1, D))


# speedup vs baseline: 1.0010x; 1.0010x over previous
import functools

import jax
import jax.numpy as jnp
from jax import lax
from jax.experimental import pallas as pl
from jax.experimental.pallas import tpu as pltpu
from jax.experimental.pallas import tpu_sc as plsc

N = 10000
E = 320000
D = 128
NC = 2
NS = 16
NW = NC * NS
EPW = E // NW
K = 128
FULL = EPW // K
TAIL = EPW - FULL * K
RPT = N // NS

BM = 1000
GRID = N // BM

_MESH = plsc.VectorSubcoreMesh(
    core_axis_name="c", subcore_axis_name="s", num_cores=NC, num_subcores=NS
)

_DEG_CHUNK = 632
_DEG_LAST = N - (NS - 1) * _DEG_CHUNK


@functools.partial(
    pl.kernel,
    mesh=_MESH,
    out_type=jax.ShapeDtypeStruct((NC * N,), jnp.float32),
    scratch_types=[
        pltpu.VMEM((K,), jnp.float32),
        pltpu.VMEM((K,), jnp.int32),
        pltpu.VMEM((TAIL,), jnp.int32),
        pltpu.VMEM((_DEG_CHUNK + 8,), jnp.float32),
        pltpu.VMEM_SHARED((N,), jnp.float32),
        pltpu.SemaphoreType.DMA,
    ],
)
def _deg_kernel(dst_hbm, out_hbm, ones_v, dst_v, dst_t, zbuf, acc_sh, sem):
    c = lax.axis_index("c")
    s = lax.axis_index("s")
    wid = c * NS + s
    for j in range(K // 16):
        ones_v[pl.ds(j * 16, 16)] = jnp.ones((16,), jnp.float32)
    for j in range((_DEG_CHUNK + 8) // 16):
        zbuf[pl.ds(j * 16, 16)] = jnp.zeros((16,), jnp.float32)

    @pl.when(s < NS - 1)
    def _():
        o = s * _DEG_CHUNK
        pltpu.sync_copy(zbuf.at[pl.ds(0, _DEG_CHUNK)], acc_sh.at[pl.ds(o, _DEG_CHUNK)])

    @pl.when(s == NS - 1)
    def _():
        o = (NS - 1) * _DEG_CHUNK
        pltpu.sync_copy(zbuf.at[pl.ds(0, _DEG_LAST)], acc_sh.at[pl.ds(o, _DEG_LAST)])

    plsc.subcore_barrier()
    base = pl.multiple_of(wid * EPW, 8)

    @pl.loop(0, FULL)
    def _(g):
        off = pl.multiple_of(base + g * K, 8)
        pltpu.sync_copy(dst_hbm.at[pl.ds(off, K)], dst_v)
        pltpu.sync_copy(ones_v, acc_sh.at[dst_v], add=True)

    offt = pl.multiple_of(base + FULL * K, 8)
    pltpu.sync_copy(dst_hbm.at[pl.ds(offt, TAIL)], dst_t)
    pltpu.sync_copy(ones_v.at[pl.ds(0, TAIL)], acc_sh.at[dst_t], add=True)
    plsc.subcore_barrier()

    cbase = pl.multiple_of(c * N, 8)

    @pl.when(s < NS - 1)
    def _():
        o = s * _DEG_CHUNK
        pltpu.sync_copy(acc_sh.at[pl.ds(o, _DEG_CHUNK)], zbuf.at[pl.ds(0, _DEG_CHUNK)])
        pltpu.sync_copy(
            zbuf.at[pl.ds(0, _DEG_CHUNK)], out_hbm.at[pl.ds(cbase + o, _DEG_CHUNK)]
        )

    @pl.when(s == NS - 1)
    def _():
        o = (NS - 1) * _DEG_CHUNK
        pltpu.sync_copy(acc_sh.at[pl.ds(o, _DEG_LAST)], zbuf.at[pl.ds(0, _DEG_LAST)])
        pltpu.sync_copy(
            zbuf.at[pl.ds(0, _DEG_LAST)], out_hbm.at[pl.ds(cbase + o, _DEG_LAST)]
        )


RANGE = N // NC
NA = 5072
EPT = E // NS
FULL2 = EPT // K
TAIL2 = EPT - FULL2 * K
SB_G = 12
SBE = SB_G * K
NSB = FULL2 // SB_G
NBUF = 2
_RW_CHUNK = 312
_RW_LAST = RANGE - (NS - 1) * _RW_CHUNK


def _remap_chunk(dst_ref, src_off, fdst_ref, j, k, lo):
    t = dst_ref[pl.ds(src_off + j * K + k * 16, 16)] - lo
    oob = plsc.bitcast(t, jnp.uint32) >= jnp.uint32(RANGE)
    trash = RANGE + (t & 63)
    fdst_ref[j, pl.ds(k * 16, 16)] = jnp.where(oob, trash, t)


@functools.partial(
    pl.kernel,
    mesh=_MESH,
    out_type=jax.ShapeDtypeStruct((NC, RANGE, D), jnp.float32),
    scratch_types=[
        [pltpu.VMEM((SBE,), jnp.int32) for _ in range(2)],
        [pltpu.VMEM((SBE,), jnp.int32) for _ in range(2)],
        [pltpu.VMEM((SB_G, K), jnp.int32) for _ in range(1)],
        pltpu.VMEM((NBUF * K, D), jnp.float32),
        pltpu.VMEM((TAIL2,), jnp.int32),
        pltpu.VMEM((TAIL2,), jnp.int32),
        pltpu.VMEM((1, TAIL2), jnp.int32),
        pltpu.VMEM((TAIL2, D), jnp.float32),
        pltpu.VMEM((_RW_LAST, D), jnp.float32),
        pltpu.VMEM_SHARED((NA, D), jnp.float32),
        pltpu.SemaphoreType.DMA,
        pltpu.SemaphoreType.DMA,
        pltpu.SemaphoreType.DMA,
    ],
)
def _agg_kernel(
    h_hbm, src_hbm, dst_hbm, out_hbm,
    srcb, dstb, fdstb, rows, src_t, dst_t, fdst_t, rows_t, bounce, acc_sh,
    gsem, sem, isem,
):
    c = lax.axis_index("c")
    s = lax.axis_index("s")
    lo = c * RANGE

    @pl.when(s < NS - 1)
    def _():
        r0 = s * _RW_CHUNK
        pltpu.sync_copy(h_hbm.at[pl.ds(lo + r0, _RW_CHUNK)], bounce.at[pl.ds(0, _RW_CHUNK)])
        pltpu.sync_copy(bounce.at[pl.ds(0, _RW_CHUNK)], acc_sh.at[pl.ds(r0, _RW_CHUNK)])

    @pl.when(s == NS - 1)
    def _():
        r0 = (NS - 1) * _RW_CHUNK
        pltpu.sync_copy(h_hbm.at[pl.ds(lo + r0, _RW_LAST)], bounce.at[pl.ds(0, _RW_LAST)])
        pltpu.sync_copy(bounce.at[pl.ds(0, _RW_LAST)], acc_sh.at[pl.ds(r0, _RW_LAST)])

    plsc.subcore_barrier()

    base = pl.multiple_of(s * EPT, 8)

    def _sb_issue(set_, sb):
        off = pl.multiple_of(base + sb * SBE, 8)
        pltpu.async_copy(src_hbm.at[pl.ds(off, SBE)], srcb[set_], isem)
        pltpu.async_copy(dst_hbm.at[pl.ds(off, SBE)], dstb[set_], isem)

    def _sb_wait(set_, sb):
        off = pl.multiple_of(base + sb * SBE, 8)
        pltpu.make_async_copy(src_hbm.at[pl.ds(off, SBE)], srcb[set_], isem).wait()
        pltpu.make_async_copy(dst_hbm.at[pl.ds(off, SBE)], dstb[set_], isem).wait()

    def _sb_exec(set_, sb):
        for j in range(SB_G):
            for k in range(K // 16):
                _remap_chunk(dstb[set_], 0, fdstb[0], j, k, lo)

        @pl.loop(0, SB_G)
        def _(j):
            b = (j % NBUF) * K
            pltpu.async_copy(
                h_hbm.at[srcb[set_].at[pl.ds(j * K, K)]], rows.at[pl.ds(b, K)], gsem
            )

            @pl.when(j > 0)
            def _():
                jm = j - 1
                bm = (jm % NBUF) * K
                pltpu.make_async_copy(
                    h_hbm.at[srcb[set_].at[pl.ds(jm * K, K)]],
                    rows.at[pl.ds(bm, K)], gsem,
                ).wait()
                pltpu.sync_copy(rows.at[pl.ds(bm, K)], acc_sh.at[fdstb[0].at[jm]], add=True)

        jl = SB_G - 1
        bl = (jl % NBUF) * K
        pltpu.make_async_copy(
            h_hbm.at[srcb[set_].at[pl.ds(jl * K, K)]], rows.at[pl.ds(bl, K)], gsem
        ).wait()
        pltpu.sync_copy(rows.at[pl.ds(bl, K)], acc_sh.at[fdstb[0].at[jl]], add=True)

    _sb_issue(0, 0)

    @pl.loop(0, (NSB - 1) // 2)
    def _(i):
        sb0 = i * 2
        _sb_wait(0, sb0)
        _sb_issue(1, sb0 + 1)
        _sb_exec(0, sb0)
        _sb_wait(1, sb0 + 1)
        _sb_issue(0, sb0 + 2)
        _sb_exec(1, sb0 + 1)

    _sb_wait(0, NSB - 1)
    _sb_exec(0, NSB - 1)

    offt = pl.multiple_of(base + FULL2 * K, 8)
    pltpu.sync_copy(src_hbm.at[pl.ds(offt, TAIL2)], src_t)
    pltpu.sync_copy(dst_hbm.at[pl.ds(offt, TAIL2)], dst_t)
    for k in range(TAIL2 // 16):
        t = dst_t[pl.ds(k * 16, 16)] - lo
        oob = plsc.bitcast(t, jnp.uint32) >= jnp.uint32(RANGE)
        trash = RANGE + (t & 63)
        fdst_t[0, pl.ds(k * 16, 16)] = jnp.where(oob, trash, t)
    pltpu.async_copy(h_hbm.at[src_t], rows_t, sem).wait()
    pltpu.sync_copy(rows_t, acc_sh.at[fdst_t.at[0]], add=True)

    plsc.subcore_barrier()

    @pl.when(s < NS - 1)
    def _():
        r0 = s * _RW_CHUNK
        pltpu.sync_copy(acc_sh.at[pl.ds(r0, _RW_CHUNK)], bounce.at[pl.ds(0, _RW_CHUNK)])
        pltpu.sync_copy(bounce.at[pl.ds(0, _RW_CHUNK)], out_hbm.at[c, pl.ds(r0, _RW_CHUNK)])

    @pl.when(s == NS - 1)
    def _():
        r0 = (NS - 1) * _RW_CHUNK
        pltpu.sync_copy(acc_sh.at[pl.ds(r0, _RW_LAST)], bounce.at[pl.ds(0, _RW_LAST)])
        pltpu.sync_copy(bounce.at[pl.ds(0, _RW_LAST)], out_hbm.at[c, pl.ds(r0, _RW_LAST)])


def _scale_body(x_ref, w_ref, degp_ref, hp_ref, dinv_ref):
    dv = lax.rsqrt(degp_ref[0] + degp_ref[1] + 1.0)
    dinv_ref[...] = dv
    hp_ref[...] = dv * jnp.dot(
        x_ref[...], w_ref[...], preferred_element_type=jnp.float32
    )


_scale = pl.pallas_call(
    _scale_body,
    grid=(GRID,),
    in_specs=[
        pl.BlockSpec((BM, D), lambda i: (i, 0)),
        pl.BlockSpec((D, D), lambda i: (0, 0)),
        pl.BlockSpec((2, BM, 1), lambda i: (0, i, 0)),
    ],
    out_specs=[
        pl.BlockSpec((BM, D), lambda i: (i, 0)),
        pl.BlockSpec((BM, 1), lambda i: (i, 0)),
    ],
    out_shape=[
        jax.ShapeDtypeStruct((N, D), jnp.float32),
        jax.ShapeDtypeStruct((N, 1), jnp.float32),
    ],
)


def _mid_body(a_ref, dinv_ref, b_ref, w_ref, o_ref):
    dv = dinv_ref[...]
    z = jnp.maximum(dv * a_ref[...] + b_ref[...], 0.0)
    o_ref[...] = dv * jnp.dot(z, w_ref[...], preferred_element_type=jnp.float32)


_mid = pl.pallas_call(
    _mid_body,
    grid=(GRID,),
    in_specs=[
        pl.BlockSpec((BM, D), lambda i: (i, 0)),
        pl.BlockSpec((BM, 1), lambda i: (i, 0)),
        pl.BlockSpec((1, D), lambda i: (0, 0)),
        pl.BlockSpec((D, D), lambda i: (0, 0)),
    ],
    out_specs=pl.BlockSpec((BM, D), lambda i: (i, 0)),
    out_shape=jax.ShapeDtypeStruct((N, D), jnp.float32),
)


def _final_body(a_ref, dinv_ref, b_ref, o_ref):
    o_ref[...] = dinv_ref[...] * a_ref[...] + b_ref[...]


_final = pl.pallas_call(
    _final_body,
    grid=(GRID,),
    in_specs=[
        pl.BlockSpec((BM, D), lambda i: (i, 0)),
        pl.BlockSpec((BM, 1), lambda i: (i, 0)),
        pl.BlockSpec((1, D), lambda i: (0, 0)),
    ],
    out_specs=pl.BlockSpec((BM, D), lambda i: (i, 0)),
    out_shape=jax.ShapeDtypeStruct((N, D), jnp.float32),
)


def kernel(x, edge_index, W1, b1, W2, b2):
    src = edge_index[0].astype(jnp.int32)
    dst = edge_index[1].astype(jnp.int32)

    degp = _deg_kernel(dst)
    h1p, dinv = _scale(x, W1, degp.reshape(2, N, 1))
    a1 = _agg_kernel(h1p, src, dst).reshape(N, D)
    h2p = _mid(a1, dinv, b1.reshape(1, D), W2)
    a2 = _agg_kernel(h2p, src, dst).reshape(N, D)
    return _final(a2, dinv, b2.reshape(1, D))
